# Initial kernel scaffold; baseline (speedup 1.0000x reference)
import jax
import jax.numpy as jnp
from jax.experimental import pallas as pl


def kernel(users, items, user_table, movie_table, W1, b1, W2, b2, Wout, bout):
    raise NotImplementedError("write your pallas kernel here")


# speedup vs baseline: 2.6783x; 2.6783x over previous
import functools

import jax
import jax.numpy as jnp
from jax import lax
from jax.experimental import pallas as pl
from jax.experimental.pallas import tpu as pltpu
from jax.experimental.pallas import tpu_sc as plsc

BATCH = 16384
EMB = 128
NC, NS = 2, 16
NW = NC * NS
B_PER_W = BATCH // NW
CHUNK = 128
NCH = B_PER_W // CHUNK


def _gather_body(users_hbm, items_hbm, utab_hbm, mtab_hbm,
                 uout_hbm, iout_hbm, idx_v, rows_v, sem):
    wid = lax.axis_index("s") * NC + lax.axis_index("c")
    base = wid * B_PER_W
    for j in range(NCH):
        pltpu.sync_copy(users_hbm.at[pl.ds(base + j * CHUNK, CHUNK)],
                        idx_v.at[j])
    copies = [pltpu.async_copy(utab_hbm.at[idx_v.at[j]], rows_v.at[j], sem)
              for j in range(NCH)]
    for c in copies:
        c.wait()
    for j in range(NCH):
        pltpu.sync_copy(rows_v.at[j],
                        uout_hbm.at[pl.ds(base + j * CHUNK, CHUNK)])
    for j in range(NCH):
        pltpu.sync_copy(items_hbm.at[pl.ds(base + j * CHUNK, CHUNK)],
                        idx_v.at[j])
    copies = [pltpu.async_copy(mtab_hbm.at[idx_v.at[j]], rows_v.at[j], sem)
              for j in range(NCH)]
    for c in copies:
        c.wait()
    for j in range(NCH):
        pltpu.sync_copy(rows_v.at[j],
                        iout_hbm.at[pl.ds(base + j * CHUNK, CHUNK)])


def _sc_gather(users, items, user_table, movie_table):
    mesh = plsc.VectorSubcoreMesh(core_axis_name="c", subcore_axis_name="s",
                                  num_cores=NC, num_subcores=NS)
    emb = jax.ShapeDtypeStruct((BATCH, EMB), jnp.float32)
    run = pl.kernel(
        _gather_body,
        mesh=mesh,
        out_type=[emb, emb],
        scratch_types=[
            pltpu.VMEM((NCH, CHUNK), jnp.int32),
            pltpu.VMEM((NCH, CHUNK, EMB), jnp.float32),
            pltpu.SemaphoreType.DMA,
        ],
    )
    return run(users, items, user_table, movie_table)


def _mlp_body(u_ref, i_ref, w1a_ref, w1b_ref, b1_ref, w2_ref, b2_ref,
              wout_ref, bout_ref, out_ref):
    h = jnp.dot(u_ref[:], w1a_ref[:], preferred_element_type=jnp.float32)
    h = h + jnp.dot(i_ref[:], w1b_ref[:], preferred_element_type=jnp.float32)
    h = jnp.maximum(h + b1_ref[:], 0.0)
    h = jnp.maximum(
        jnp.dot(h, w2_ref[:], preferred_element_type=jnp.float32) + b2_ref[:],
        0.0)
    out_ref[:] = (jnp.dot(h, wout_ref[:], preferred_element_type=jnp.float32)
                  + bout_ref[:])


def _tc_mlp(u_emb, i_emb, W1, b1, W2, b2, Wout, bout, tile=2048):
    w1a = W1[:EMB]
    w1b = W1[EMB:]
    grid = (BATCH // tile,)
    row_spec = pl.BlockSpec((tile, EMB), lambda g: (g, 0))
    full = lambda shape: pl.BlockSpec(shape, lambda g: (0,) * len(shape))
    return pl.pallas_call(
        _mlp_body,
        grid=grid,
        in_specs=[
            row_spec, row_spec,
            full((EMB, 128)), full((EMB, 128)), full((1, 128)),
            full((128, 64)), full((1, 64)),
            full((64, 1)), full((1, 1)),
        ],
        out_specs=pl.BlockSpec((tile, 1), lambda g: (g, 0)),
        out_shape=jax.ShapeDtypeStruct((BATCH, 1), jnp.float32),
    )(u_emb, i_emb, w1a, w1b, b1.reshape(1, 128), W2, b2.reshape(1, 64),
      Wout, bout.reshape(1, 1))


@jax.jit
def kernel(users, items, user_table, movie_table, W1, b1, W2, b2, Wout, bout):
    u_emb, i_emb = _sc_gather(users, items, user_table, movie_table)
    return _tc_mlp(u_emb, i_emb, W1, b1, W2, b2, Wout, bout)


# speedup vs baseline: 2.7403x; 1.0232x over previous
import functools

import jax
import jax.numpy as jnp
from jax import lax
from jax.experimental import pallas as pl
from jax.experimental.pallas import tpu as pltpu
from jax.experimental.pallas import tpu_sc as plsc

BATCH = 16384
EMB = 128
NC, NS = 2, 16
NW = NC * NS
CHUNK = 128
PIPE = 2
CB = BATCH // PIPE
B_PER_W = CB // NW
NCH = B_PER_W // CHUNK


def _gather_body(users_hbm, items_hbm, utab_hbm, mtab_hbm,
                 uout_hbm, iout_hbm, idx_v, rows_v, sem):
    wid = lax.axis_index("s") * NC + lax.axis_index("c")
    base = wid * B_PER_W
    for j in range(NCH):
        pltpu.sync_copy(users_hbm.at[pl.ds(base + j * CHUNK, CHUNK)],
                        idx_v.at[j])
    copies = [pltpu.async_copy(utab_hbm.at[idx_v.at[j]], rows_v.at[j], sem)
              for j in range(NCH)]
    for c in copies:
        c.wait()
    for j in range(NCH):
        pltpu.sync_copy(rows_v.at[j],
                        uout_hbm.at[pl.ds(base + j * CHUNK, CHUNK)])
    for j in range(NCH):
        pltpu.sync_copy(items_hbm.at[pl.ds(base + j * CHUNK, CHUNK)],
                        idx_v.at[j])
    copies = [pltpu.async_copy(mtab_hbm.at[idx_v.at[j]], rows_v.at[j], sem)
              for j in range(NCH)]
    for c in copies:
        c.wait()
    for j in range(NCH):
        pltpu.sync_copy(rows_v.at[j],
                        iout_hbm.at[pl.ds(base + j * CHUNK, CHUNK)])


def _sc_gather(users, items, user_table, movie_table):
    mesh = plsc.VectorSubcoreMesh(core_axis_name="c", subcore_axis_name="s",
                                  num_cores=NC, num_subcores=NS)
    emb = jax.ShapeDtypeStruct((CB, EMB), jnp.float32)
    run = pl.kernel(
        _gather_body,
        mesh=mesh,
        out_type=[emb, emb],
        scratch_types=[
            pltpu.VMEM((NCH, CHUNK), jnp.int32),
            pltpu.VMEM((NCH, CHUNK, EMB), jnp.float32),
            pltpu.SemaphoreType.DMA,
        ],
    )
    return run(users, items, user_table, movie_table)


def _mlp_body(u_ref, i_ref, w1a_ref, w1b_ref, b1_ref, w2_ref, b2_ref,
              wout_ref, bout_ref, out_ref):
    h = jnp.dot(u_ref[:], w1a_ref[:], preferred_element_type=jnp.float32)
    h = h + jnp.dot(i_ref[:], w1b_ref[:], preferred_element_type=jnp.float32)
    h = jnp.maximum(h + b1_ref[:], 0.0)
    h = jnp.maximum(
        jnp.dot(h, w2_ref[:], preferred_element_type=jnp.float32) + b2_ref[:],
        0.0)
    out_ref[:] = (jnp.dot(h, wout_ref[:], preferred_element_type=jnp.float32)
                  + bout_ref[:])


def _tc_mlp(u_emb, i_emb, w1a, w1b, b1, W2, b2, Wout, bout, tile=2048):
    grid = (CB // tile,)
    row_spec = pl.BlockSpec((tile, EMB), lambda g: (g, 0))
    full = lambda shape: pl.BlockSpec(shape, lambda g: (0,) * len(shape))
    return pl.pallas_call(
        _mlp_body,
        grid=grid,
        in_specs=[
            row_spec, row_spec,
            full((EMB, 128)), full((EMB, 128)), full((1, 128)),
            full((128, 64)), full((1, 64)),
            full((64, 1)), full((1, 1)),
        ],
        out_specs=pl.BlockSpec((tile, 1), lambda g: (g, 0)),
        out_shape=jax.ShapeDtypeStruct((CB, 1), jnp.float32),
    )(u_emb, i_emb, w1a, w1b, b1, W2, b2, Wout, bout)


@jax.jit
def kernel(users, items, user_table, movie_table, W1, b1, W2, b2, Wout, bout):
    w1a, w1b = W1[:EMB], W1[EMB:]
    b1r, b2r, boutr = b1.reshape(1, 128), b2.reshape(1, 64), bout.reshape(1, 1)
    embs = [_sc_gather(users[p * CB:(p + 1) * CB], items[p * CB:(p + 1) * CB],
                       user_table, movie_table)
            for p in range(PIPE)]
    outs = [_tc_mlp(u, i, w1a, w1b, b1r, W2, b2r, Wout, boutr)
            for u, i in embs]
    return jnp.concatenate(outs, axis=0)
